# Initial kernel scaffold; baseline (speedup 1.0000x reference)
import functools

import jax
import jax.numpy as jnp
from jax import lax
from jax.experimental import pallas as pl
from jax.experimental.pallas import tpu as pltpu
from jax.experimental.pallas import tpu_sc as plsc

F = 128
NCORE = 2
NTILE = 16
WE = 128
G = 1000
GP = 1024


def _rup(n, m):
    return -(-n // m) * m


def _pad1(x, n, val):
    return jnp.concatenate([x, jnp.full((n - x.shape[0],), val, x.dtype)])


def _dst_cfg(n):
    ch = 12544 if n >= 20000 else 2560
    nch = max(2, _rup(-(-n // ch), 2))
    return nch * ch, ch


def _sc_agg(feats, src, dst, n_dst_p, ch, zeros_h):
    ep = src.shape[0]
    per_tile = ep // NTILE
    n_win = per_tile // WE
    n_chunks = n_dst_p // ch
    cpc = n_chunks // NCORE
    acc_rows = ch + 256
    ptr = acc_rows // NTILE
    zr = zeros_h.shape[0]
    nseg = -(-ptr // zr)
    wpt = ch // NTILE

    mesh = plsc.VectorSubcoreMesh(core_axis_name="c", subcore_axis_name="s")

    @functools.partial(
        pl.kernel, mesh=mesh,
        out_type=jax.ShapeDtypeStruct((n_dst_p, F), jnp.float32),
        scratch_types=[
            pltpu.VMEM_SHARED((acc_rows, F), jnp.float32),
            pltpu.VMEM((1, WE), jnp.int32),
            pltpu.VMEM((1, WE), jnp.int32),
            pltpu.VMEM((WE, F), jnp.float32),
            pltpu.VMEM((zeros_h.shape[0], F), jnp.float32),
            pltpu.SemaphoreType.DMA,
        ])
    def k(feats_h, src_h, dst_h, z_h, out_h, acc_sh, sidx_v, lidx_v, rows_v,
          zbuf_v, sem):
        c = lax.axis_index("c")
        s = lax.axis_index("s")
        lane = lax.iota(jnp.int32, 16)
        pltpu.sync_copy(z_h, zbuf_v)
        for ci in range(cpc):
            chunk = ci * NCORE + c
            base = chunk * ch
            for i in range(nseg):
                st = s * ptr + min(i * zr, ptr - zr)
                pltpu.sync_copy(zbuf_v, acc_sh.at[pl.ds(st, zr)])
            plsc.subcore_barrier()
            e0 = s * per_tile
            dump0 = ch + s * 16

            def _win(w, _):
                off = e0 + w * WE
                pltpu.sync_copy(src_h.at[pl.ds(off, WE)], sidx_v.at[0])
                pltpu.sync_copy(dst_h.at[pl.ds(off, WE)], lidx_v.at[0])
                for g in range(WE // 16):
                    dv = lidx_v[0, pl.ds(g * 16, 16)]
                    inm = (dv >= base) & (dv < base + ch)
                    lv = jnp.where(inm, dv - base, dump0 + lane)
                    lidx_v[0, pl.ds(g * 16, 16)] = lv
                pltpu.async_copy(feats_h.at[sidx_v.at[0]], rows_v, sem).wait()
                pltpu.sync_copy(rows_v, acc_sh.at[lidx_v.at[0]], add=True)
                return 0

            lax.fori_loop(0, n_win, _win, 0)
            plsc.subcore_barrier()
            pltpu.sync_copy(acc_sh.at[pl.ds(s * wpt, wpt)],
                            out_h.at[pl.ds(base + s * wpt, wpt)])
            plsc.subcore_barrier()

    return k(feats, src, dst, zeros_h)


def _sc_pool(fa, ba, fb, bb, fm, bm, z128, z16, ones16):
    mesh = plsc.VectorSubcoreMesh(core_axis_name="c", subcore_axis_name="s")
    sizes = (fa.shape[0], fb.shape[0], fm.shape[0])

    @functools.partial(
        pl.kernel, mesh=mesh,
        out_type=(jax.ShapeDtypeStruct((NCORE * GP, F), jnp.float32),
                  jax.ShapeDtypeStruct((NCORE * GP, 16), jnp.float32)),
        scratch_types=[
            pltpu.VMEM_SHARED((GP, F), jnp.float32),
            pltpu.VMEM_SHARED((GP, 16), jnp.float32),
            pltpu.VMEM((1, WE), jnp.int32),
            pltpu.VMEM((WE, F), jnp.float32),
            pltpu.VMEM((WE, 16), jnp.float32),
            pltpu.SemaphoreType.DMA,
        ])
    def k(fa_h, ba_h, fb_h, bb_h, fm_h, bm_h, z128_h, z16_h, ones_h,
          gsum_h, gcnt_h, acc_sh, cacc_sh, idx_v, rows_v, ones_v, sem):
        c = lax.axis_index("c")
        s = lax.axis_index("s")
        wid = s * NCORE + c
        spt = GP // NTILE
        pltpu.sync_copy(z128_h, acc_sh.at[pl.ds(s * spt, spt)])
        pltpu.sync_copy(z16_h, cacc_sh.at[pl.ds(s * spt, spt)])
        pltpu.sync_copy(ones_h, ones_v)
        plsc.subcore_barrier()
        for f_h, b_h, np_ in ((fa_h, ba_h, sizes[0]), (fb_h, bb_h, sizes[1]),
                              (fm_h, bm_h, sizes[2])):
            tot_w = np_ // WE
            nw_max = -(-tot_w // (NCORE * NTILE))

            def _win(j, _, f_h=f_h, b_h=b_h, tot_w=tot_w):
                gw = j * (NCORE * NTILE) + wid

                @pl.when(gw < tot_w)
                def _():
                    off = gw * WE
                    pltpu.sync_copy(b_h.at[pl.ds(off, WE)], idx_v.at[0])
                    pltpu.sync_copy(f_h.at[pl.ds(off, WE)], rows_v)
                    pltpu.sync_copy(rows_v, acc_sh.at[idx_v.at[0]], add=True)
                    pltpu.sync_copy(ones_v, cacc_sh.at[idx_v.at[0]], add=True)

                return 0

            lax.fori_loop(0, nw_max, _win, 0)
        plsc.subcore_barrier()
        pltpu.sync_copy(acc_sh.at[pl.ds(s * spt, spt)],
                        gsum_h.at[pl.ds(c * GP + s * spt, spt)])
        pltpu.sync_copy(cacc_sh.at[pl.ds(s * spt, spt)],
                        gcnt_h.at[pl.ds(c * GP + s * spt, spt)])

    return k(fa, ba, fb, bb, fm, bm, z128, z16, ones16)


def _tc_embed(codes, table):
    n_pad = codes.shape[0]
    br = 1024
    nb = n_pad // br
    cp = table.shape[0]
    c3 = codes.reshape(nb, 1, br)

    def body(c_ref, t_ref, o_ref):
        cod = c_ref[0, 0, :]
        oh = (cod[:, None] == lax.broadcasted_iota(jnp.int32, (br, cp), 1)
              ).astype(jnp.float32)
        o_ref[...] = jnp.dot(oh, t_ref[...], preferred_element_type=jnp.float32)

    return pl.pallas_call(
        body, grid=(nb,),
        in_specs=[pl.BlockSpec((1, 1, br), lambda i: (i, 0, 0)),
                  pl.BlockSpec((cp, F), lambda i: (0, 0))],
        out_specs=pl.BlockSpec((br, F), lambda i: (i, 0)),
        out_shape=jax.ShapeDtypeStruct((n_pad, F), jnp.float32),
    )(c3, table)


def _tc_gin_h(xd, agg, wmat, aux, n_real):
    ndp = xd.shape[0]
    br = 512
    nb = ndp // br

    def body(x_ref, g_ref, w_ref, aux_ref, h_ref, st_ref):
        i = pl.program_id(0)

        @pl.when(i == 0)
        def _():
            st_ref[...] = jnp.zeros_like(st_ref)

        z = jnp.dot(x_ref[...] + g_ref[...], w_ref[...],
                    preferred_element_type=jnp.float32) + aux_ref[0:1, :]
        h = jnp.maximum(z, 0.0) + aux_ref[1:2, :] * jnp.minimum(z, 0.0)
        h_ref[...] = h
        rid = i * br + lax.broadcasted_iota(jnp.int32, (br, 1), 0)
        hm = h * (rid < n_real).astype(jnp.float32)
        s0 = jnp.sum(hm, axis=0, keepdims=True)
        s1 = jnp.sum(hm * hm, axis=0, keepdims=True)
        st_ref[...] += jnp.concatenate(
            [s0, s1, jnp.zeros((6, F), jnp.float32)], axis=0)

    return pl.pallas_call(
        body, grid=(nb,),
        in_specs=[pl.BlockSpec((br, F), lambda i: (i, 0)),
                  pl.BlockSpec((br, F), lambda i: (i, 0)),
                  pl.BlockSpec((F, F), lambda i: (0, 0)),
                  pl.BlockSpec((8, F), lambda i: (0, 0))],
        out_specs=[pl.BlockSpec((br, F), lambda i: (i, 0)),
                   pl.BlockSpec((8, F), lambda i: (0, 0))],
        out_shape=[jax.ShapeDtypeStruct((ndp, F), jnp.float32),
                   jax.ShapeDtypeStruct((8, F), jnp.float32)],
    )(xd, agg, wmat, aux)


def _tc_affine1(h1, st1):
    ndp = h1.shape[0]
    br = 512

    def body(a_ref, sa_ref, o_ref):
        o_ref[...] = a_ref[...] * sa_ref[0:1, :] + sa_ref[1:2, :]

    return pl.pallas_call(
        body, grid=(ndp // br,),
        in_specs=[pl.BlockSpec((br, F), lambda i: (i, 0)),
                  pl.BlockSpec((8, F), lambda i: (0, 0))],
        out_specs=pl.BlockSpec((br, F), lambda i: (i, 0)),
        out_shape=jax.ShapeDtypeStruct((ndp, F), jnp.float32),
    )(h1, st1)


def _tc_affine2(h1, st1, h2, st2):
    ndp = h1.shape[0]
    br = 512

    def body(a_ref, sa_ref, b_ref, sb_ref, o_ref):
        o_ref[...] = (a_ref[...] * sa_ref[0:1, :] + sa_ref[1:2, :]
                      + b_ref[...] * sb_ref[0:1, :] + sb_ref[1:2, :])

    return pl.pallas_call(
        body, grid=(ndp // br,),
        in_specs=[pl.BlockSpec((br, F), lambda i: (i, 0)),
                  pl.BlockSpec((8, F), lambda i: (0, 0)),
                  pl.BlockSpec((br, F), lambda i: (i, 0)),
                  pl.BlockSpec((8, F), lambda i: (0, 0))],
        out_specs=pl.BlockSpec((br, F), lambda i: (i, 0)),
        out_shape=jax.ShapeDtypeStruct((ndp, F), jnp.float32),
    )(h1, st1, h2, st2)


def _tc_head(gsum, gcnt, w1p, w2p, aux):
    def body(gs_ref, gc_ref, w1_ref, w2_ref, aux_ref, o_ref):
        gsum2 = gs_ref[0:GP, :] + gs_ref[GP:2 * GP, :]
        cnt = gc_ref[0:GP, :] + gc_ref[GP:2 * GP, :]
        c1 = jnp.sum(cnt, axis=1, keepdims=True) * (1.0 / 16.0)
        g = gsum2 / jnp.maximum(c1, 1.0)
        z1 = jnp.dot(g, w1_ref[...],
                     preferred_element_type=jnp.float32) + aux_ref[0:1, :]
        h1 = jnp.maximum(z1, 0.0) + aux_ref[1:2, :] * jnp.minimum(z1, 0.0)
        o_ref[...] = jnp.dot(h1, w2_ref[...],
                             preferred_element_type=jnp.float32) + aux_ref[2:3, :]

    return pl.pallas_call(
        body,
        out_shape=jax.ShapeDtypeStruct((GP, F), jnp.float32),
    )(gsum, gcnt, w1p, w2p, aux)


def _bn_affine(st, n, gamma, beta):
    mu = st[0] / n
    var = st[1] / n - mu * mu
    sc = gamma * lax.rsqrt(var + 1e-5)
    tt = beta - mu * sc
    return jnp.zeros((8, F), jnp.float32).at[0].set(sc).at[1].set(tt)


def _aux_pb(b, a):
    return (jnp.zeros((8, F), jnp.float32)
            .at[0].set(b).at[1].set(jnp.full((F,), a, jnp.float32)))


def kernel(x_atoms, x_bonds, x_monosacchs, ei_aa, ei_ab, ei_bb, ei_bm, ei_mm,
           batch_atoms, batch_bonds, batch_monosacchs, params):
    na, nb_, nm = x_atoms.shape[0], x_bonds.shape[0], x_monosacchs.shape[0]
    nap, ch_a = _dst_cfg(na)
    nbp, ch_b = _dst_cfg(nb_)
    nmp, ch_m = _dst_cfg(nm)
    n_real = {"a": na, "b": nb_, "m": nm}
    n_pad = {"a": nap, "b": nbp, "m": nmp}
    ch = {"a": ch_a, "b": ch_b, "m": ch_m}

    zeros_h = jnp.zeros((160, F), jnp.float32)
    z128 = jnp.zeros((GP // NTILE, F), jnp.float32)
    z16 = jnp.zeros((GP // NTILE, 16), jnp.float32)
    ones16 = jnp.ones((WE, 16), jnp.float32)

    ta = jnp.pad(params["atom_emb"], ((0, 8), (0, 0)))
    tb = params["bond_emb"]
    tm = params["mono_emb"]
    feats = {
        "a": _tc_embed(_pad1(x_atoms.astype(jnp.int32), nap, 0), ta),
        "b": _tc_embed(_pad1(x_bonds.astype(jnp.int32), nbp, 0), tb),
        "m": _tc_embed(_pad1(x_monosacchs.astype(jnp.int32), nmp, 0), tm),
    }

    def _edges(ei):
        ep = _rup(ei.shape[1], NTILE * WE)
        return (_pad1(ei[0].astype(jnp.int32), ep, 0),
                _pad1(ei[1].astype(jnp.int32), ep, -1))

    rels = {
        "aa": (_edges(ei_aa), "a", "a"),
        "ab": (_edges(ei_ab), "a", "b"),
        "bb": (_edges(ei_bb), "b", "b"),
        "bm": (_edges(ei_bm), "b", "m"),
        "mm": (_edges(ei_mm), "m", "m"),
    }

    for l in range(3):
        p = params["convs"][l]
        h, st = {}, {}
        for r, ((src, dst), ks, kd) in rels.items():
            agg = _sc_agg(feats[ks], src, dst, n_pad[kd], ch[kd], zeros_h)
            h[r], stats = _tc_gin_h(feats[kd], agg,
                                    p[r]["W"], _aux_pb(p[r]["b"], p[r]["a"]),
                                    n_real[kd])
            st[r] = _bn_affine(stats, n_real[kd], p[r]["gamma"], p[r]["beta"])
        feats = {
            "a": _tc_affine1(h["aa"], st["aa"]),
            "b": _tc_affine2(h["ab"], st["ab"], h["bb"], st["bb"]),
            "m": _tc_affine2(h["bm"], st["bm"], h["mm"], st["mm"]),
        }

    pad_a = G + 8 + (jnp.arange(nap - na, dtype=jnp.int32) % 16)
    pad_b = G + 8 + (jnp.arange(nbp - nb_, dtype=jnp.int32) % 16)
    pad_m = G + 8 + (jnp.arange(nmp - nm, dtype=jnp.int32) % 16)
    bap = jnp.concatenate([batch_atoms.astype(jnp.int32), pad_a])
    bbp = jnp.concatenate([batch_bonds.astype(jnp.int32), pad_b])
    bmp = jnp.concatenate([batch_monosacchs.astype(jnp.int32), pad_m])

    gsum, gcnt = _sc_pool(feats["a"], bap, feats["b"], bbp, feats["m"], bmp,
                          z128, z16, ones16)

    hd = params["head"]
    w1p = jnp.pad(hd["l1"]["W"], ((0, 0), (0, F - hd["l1"]["W"].shape[1])))
    w2p = jnp.pad(hd["l2"]["W"], ((0, F - hd["l2"]["W"].shape[0]),
                                  (0, F - hd["l2"]["W"].shape[1])))
    auxh = (jnp.zeros((8, F), jnp.float32)
            .at[0, 0:hd["l1"]["b"].shape[0]].set(hd["l1"]["b"])
            .at[1].set(jnp.full((F,), hd["a"], jnp.float32))
            .at[2, 0:hd["l2"]["b"].shape[0]].set(hd["l2"]["b"]))
    pred = _tc_head(gsum, gcnt, w1p, w2p, auxh)
    return pred[:G, 0]


# speedup vs baseline: 1.1427x; 1.1427x over previous
import functools

import jax
import jax.numpy as jnp
from jax import lax
from jax.experimental import pallas as pl
from jax.experimental.pallas import tpu as pltpu
from jax.experimental.pallas import tpu_sc as plsc

F = 128
NCORE = 2
NTILE = 16
WE = 128
G = 1000
GP = 1024


def _rup(n, m):
    return -(-n // m) * m


def _pad1(x, n, val):
    return jnp.concatenate([x, jnp.full((n - x.shape[0],), val, x.dtype)])


def _dst_cfg(n):
    ch = 12544 if n >= 20000 else 2560
    nch = max(2, _rup(-(-n // ch), 2))
    return nch * ch, ch


def _sc_agg(feats, src, dst, n_dst_p, ch, zeros_h):
    ep = src.shape[0]
    per_tile = ep // NTILE
    n_win = per_tile // WE
    n_chunks = n_dst_p // ch
    cpc = n_chunks // NCORE
    acc_rows = ch + 256
    ptr = acc_rows // NTILE
    zr = zeros_h.shape[0]
    nseg = -(-ptr // zr)
    wpt = ch // NTILE

    mesh = plsc.VectorSubcoreMesh(core_axis_name="c", subcore_axis_name="s")

    @functools.partial(
        pl.kernel, mesh=mesh,
        out_type=jax.ShapeDtypeStruct((n_dst_p, F), jnp.float32),
        scratch_types=[
            pltpu.VMEM_SHARED((acc_rows, F), jnp.float32),
            pltpu.VMEM((1, WE), jnp.int32),
            pltpu.VMEM((1, WE), jnp.int32),
            pltpu.VMEM((WE, F), jnp.float32),
            pltpu.VMEM((zeros_h.shape[0], F), jnp.float32),
            pltpu.SemaphoreType.DMA,
        ])
    def k(feats_h, src_h, dst_h, z_h, out_h, acc_sh, sidx_v, lidx_v, rows_v,
          zbuf_v, sem):
        c = lax.axis_index("c")
        s = lax.axis_index("s")
        lane = lax.iota(jnp.int32, 16)
        pltpu.sync_copy(z_h, zbuf_v)
        for ci in range(cpc):
            chunk = ci * NCORE + c
            base = chunk * ch
            for i in range(nseg):
                st = s * ptr + min(i * zr, ptr - zr)
                pltpu.sync_copy(zbuf_v, acc_sh.at[pl.ds(st, zr)])
            plsc.subcore_barrier()
            e0 = s * per_tile
            dump0 = ch + s * 16

            def _win(w, _):
                off = e0 + w * WE
                pltpu.sync_copy(src_h.at[pl.ds(off, WE)], sidx_v.at[0])
                pltpu.sync_copy(dst_h.at[pl.ds(off, WE)], lidx_v.at[0])
                for g in range(WE // 16):
                    dv = lidx_v[0, pl.ds(g * 16, 16)]
                    inm = (dv >= base) & (dv < base + ch)
                    lv = jnp.where(inm, dv - base, dump0 + lane)
                    lidx_v[0, pl.ds(g * 16, 16)] = lv
                pltpu.async_copy(feats_h.at[sidx_v.at[0]], rows_v, sem).wait()
                pltpu.sync_copy(rows_v, acc_sh.at[lidx_v.at[0]], add=True)
                return 0

            lax.fori_loop(0, n_win, _win, 0)
            plsc.subcore_barrier()
            pltpu.sync_copy(acc_sh.at[pl.ds(s * wpt, wpt)],
                            out_h.at[pl.ds(base + s * wpt, wpt)])
            plsc.subcore_barrier()

    return k(feats, src, dst, zeros_h)


def _sc_pool(fa, ba, fb, bb, fm, bm, z128, z16, ones16):
    mesh = plsc.VectorSubcoreMesh(core_axis_name="c", subcore_axis_name="s")
    sizes = (fa.shape[0], fb.shape[0], fm.shape[0])

    @functools.partial(
        pl.kernel, mesh=mesh,
        out_type=(jax.ShapeDtypeStruct((NCORE * GP, F), jnp.float32),
                  jax.ShapeDtypeStruct((NCORE * GP, F), jnp.float32)),
        scratch_types=[
            pltpu.VMEM_SHARED((GP, F), jnp.float32),
            pltpu.VMEM_SHARED((GP, F), jnp.float32),
            pltpu.VMEM((1, WE), jnp.int32),
            pltpu.VMEM((WE, F), jnp.float32),
            pltpu.VMEM((WE, F), jnp.float32),
            pltpu.SemaphoreType.DMA,
        ])
    def k(fa_h, ba_h, fb_h, bb_h, fm_h, bm_h, z128_h, z16_h, ones_h,
          gsum_h, gcnt_h, acc_sh, cacc_sh, idx_v, rows_v, ones_v, sem):
        c = lax.axis_index("c")
        s = lax.axis_index("s")
        wid = s * NCORE + c
        spt = GP // NTILE
        pltpu.sync_copy(z128_h, acc_sh.at[pl.ds(s * spt, spt)])
        pltpu.sync_copy(z128_h, cacc_sh.at[pl.ds(s * spt, spt)])
        pltpu.sync_copy(ones_h, ones_v)
        plsc.subcore_barrier()
        for f_h, b_h, np_ in ((fa_h, ba_h, sizes[0]), (fb_h, bb_h, sizes[1]),
                              (fm_h, bm_h, sizes[2])):
            tot_w = np_ // WE
            nw_max = -(-tot_w // (NCORE * NTILE))

            def _win(j, _, f_h=f_h, b_h=b_h, tot_w=tot_w):
                gw = j * (NCORE * NTILE) + wid

                @pl.when(gw < tot_w)
                def _():
                    off = gw * WE
                    pltpu.sync_copy(b_h.at[pl.ds(off, WE)], idx_v.at[0])
                    pltpu.sync_copy(f_h.at[pl.ds(off, WE)], rows_v)
                    pltpu.sync_copy(rows_v, acc_sh.at[idx_v.at[0]], add=True)
                    pltpu.sync_copy(ones_v, cacc_sh.at[idx_v.at[0]], add=True)

                return 0

            lax.fori_loop(0, nw_max, _win, 0)
        plsc.subcore_barrier()
        pltpu.sync_copy(acc_sh.at[pl.ds(s * spt, spt)],
                        gsum_h.at[pl.ds(c * GP + s * spt, spt)])
        pltpu.sync_copy(cacc_sh.at[pl.ds(s * spt, spt)],
                        gcnt_h.at[pl.ds(c * GP + s * spt, spt)])

    return k(fa, ba, fb, bb, fm, bm, z128, z16, ones16)


def _tc_embed(codes, table):
    n_pad = codes.shape[0]
    br = 1024
    nb = n_pad // br
    cp = table.shape[0]
    c3 = codes.reshape(nb, 1, br)

    def body(c_ref, t_ref, o_ref):
        cod = c_ref[0, 0, :]
        oh = (cod[:, None] == lax.broadcasted_iota(jnp.int32, (br, cp), 1)
              ).astype(jnp.float32)
        o_ref[...] = jnp.dot(oh, t_ref[...], preferred_element_type=jnp.float32)

    return pl.pallas_call(
        body, grid=(nb,),
        in_specs=[pl.BlockSpec((1, 1, br), lambda i: (i, 0, 0)),
                  pl.BlockSpec((cp, F), lambda i: (0, 0))],
        out_specs=pl.BlockSpec((br, F), lambda i: (i, 0)),
        out_shape=jax.ShapeDtypeStruct((n_pad, F), jnp.float32),
    )(c3, table)


def _tc_gin_h(xd, agg, wmat, aux, n_real):
    ndp = xd.shape[0]
    br = 512
    nb = ndp // br

    def body(x_ref, g_ref, w_ref, aux_ref, h_ref, st_ref):
        i = pl.program_id(0)

        @pl.when(i == 0)
        def _():
            st_ref[...] = jnp.zeros_like(st_ref)

        z = jnp.dot(x_ref[...] + g_ref[...], w_ref[...],
                    preferred_element_type=jnp.float32) + aux_ref[0:1, :]
        h = jnp.maximum(z, 0.0) + aux_ref[1:2, :] * jnp.minimum(z, 0.0)
        h_ref[...] = h
        rid = i * br + lax.broadcasted_iota(jnp.int32, (br, 1), 0)
        hm = h * (rid < n_real).astype(jnp.float32)
        s0 = jnp.sum(hm, axis=0, keepdims=True)
        s1 = jnp.sum(hm * hm, axis=0, keepdims=True)
        st_ref[...] += jnp.concatenate(
            [s0, s1, jnp.zeros((6, F), jnp.float32)], axis=0)

    return pl.pallas_call(
        body, grid=(nb,),
        in_specs=[pl.BlockSpec((br, F), lambda i: (i, 0)),
                  pl.BlockSpec((br, F), lambda i: (i, 0)),
                  pl.BlockSpec((F, F), lambda i: (0, 0)),
                  pl.BlockSpec((8, F), lambda i: (0, 0))],
        out_specs=[pl.BlockSpec((br, F), lambda i: (i, 0)),
                   pl.BlockSpec((8, F), lambda i: (0, 0))],
        out_shape=[jax.ShapeDtypeStruct((ndp, F), jnp.float32),
                   jax.ShapeDtypeStruct((8, F), jnp.float32)],
    )(xd, agg, wmat, aux)


def _tc_affine1(h1, st1):
    ndp = h1.shape[0]
    br = 512

    def body(a_ref, sa_ref, o_ref):
        o_ref[...] = a_ref[...] * sa_ref[0:1, :] + sa_ref[1:2, :]

    return pl.pallas_call(
        body, grid=(ndp // br,),
        in_specs=[pl.BlockSpec((br, F), lambda i: (i, 0)),
                  pl.BlockSpec((8, F), lambda i: (0, 0))],
        out_specs=pl.BlockSpec((br, F), lambda i: (i, 0)),
        out_shape=jax.ShapeDtypeStruct((ndp, F), jnp.float32),
    )(h1, st1)


def _tc_affine2(h1, st1, h2, st2):
    ndp = h1.shape[0]
    br = 512

    def body(a_ref, sa_ref, b_ref, sb_ref, o_ref):
        o_ref[...] = (a_ref[...] * sa_ref[0:1, :] + sa_ref[1:2, :]
                      + b_ref[...] * sb_ref[0:1, :] + sb_ref[1:2, :])

    return pl.pallas_call(
        body, grid=(ndp // br,),
        in_specs=[pl.BlockSpec((br, F), lambda i: (i, 0)),
                  pl.BlockSpec((8, F), lambda i: (0, 0)),
                  pl.BlockSpec((br, F), lambda i: (i, 0)),
                  pl.BlockSpec((8, F), lambda i: (0, 0))],
        out_specs=pl.BlockSpec((br, F), lambda i: (i, 0)),
        out_shape=jax.ShapeDtypeStruct((ndp, F), jnp.float32),
    )(h1, st1, h2, st2)


def _tc_head(gsum, gcnt, w1p, w2p, aux):
    def body(gs_ref, gc_ref, w1_ref, w2_ref, aux_ref, o_ref):
        gsum2 = gs_ref[0:GP, :] + gs_ref[GP:2 * GP, :]
        cnt = gc_ref[0:GP, :] + gc_ref[GP:2 * GP, :]
        c1 = jnp.sum(cnt, axis=1, keepdims=True) * (1.0 / F)
        g = gsum2 / jnp.maximum(c1, 1.0)
        z1 = jnp.dot(g, w1_ref[...],
                     preferred_element_type=jnp.float32) + aux_ref[0:1, :]
        h1 = jnp.maximum(z1, 0.0) + aux_ref[1:2, :] * jnp.minimum(z1, 0.0)
        o_ref[...] = jnp.dot(h1, w2_ref[...],
                             preferred_element_type=jnp.float32) + aux_ref[2:3, :]

    return pl.pallas_call(
        body,
        out_shape=jax.ShapeDtypeStruct((GP, F), jnp.float32),
    )(gsum, gcnt, w1p, w2p, aux)


def _bn_affine(st, n, gamma, beta):
    mu = st[0] / n
    var = st[1] / n - mu * mu
    sc = gamma * lax.rsqrt(var + 1e-5)
    tt = beta - mu * sc
    return jnp.zeros((8, F), jnp.float32).at[0].set(sc).at[1].set(tt)


def _aux_pb(b, a):
    return (jnp.zeros((8, F), jnp.float32)
            .at[0].set(b).at[1].set(jnp.full((F,), a, jnp.float32)))


def kernel(x_atoms, x_bonds, x_monosacchs, ei_aa, ei_ab, ei_bb, ei_bm, ei_mm,
           batch_atoms, batch_bonds, batch_monosacchs, params):
    na, nb_, nm = x_atoms.shape[0], x_bonds.shape[0], x_monosacchs.shape[0]
    nap, ch_a = _dst_cfg(na)
    nbp, ch_b = _dst_cfg(nb_)
    nmp, ch_m = _dst_cfg(nm)
    n_real = {"a": na, "b": nb_, "m": nm}
    n_pad = {"a": nap, "b": nbp, "m": nmp}
    ch = {"a": ch_a, "b": ch_b, "m": ch_m}

    zeros_h = jnp.zeros((32, F), jnp.float32)
    z128 = jnp.zeros((GP // NTILE, F), jnp.float32)
    z16 = jnp.zeros((GP // NTILE, 16), jnp.float32)
    ones16 = jnp.ones((WE, F), jnp.float32)

    ta = jnp.pad(params["atom_emb"], ((0, 8), (0, 0)))
    tb = params["bond_emb"]
    tm = params["mono_emb"]
    feats = {
        "a": _tc_embed(_pad1(x_atoms.astype(jnp.int32), nap, 0), ta),
        "b": _tc_embed(_pad1(x_bonds.astype(jnp.int32), nbp, 0), tb),
        "m": _tc_embed(_pad1(x_monosacchs.astype(jnp.int32), nmp, 0), tm),
    }

    def _edges(ei):
        ep = _rup(ei.shape[1], NTILE * WE)
        return (_pad1(ei[0].astype(jnp.int32), ep, 0),
                _pad1(ei[1].astype(jnp.int32), ep, -1))

    rels = {
        "aa": (_edges(ei_aa), "a", "a"),
        "ab": (_edges(ei_ab), "a", "b"),
        "bb": (_edges(ei_bb), "b", "b"),
        "bm": (_edges(ei_bm), "b", "m"),
        "mm": (_edges(ei_mm), "m", "m"),
    }

    for l in range(3):
        p = params["convs"][l]
        h, st = {}, {}
        for r, ((src, dst), ks, kd) in rels.items():
            agg = _sc_agg(feats[ks], src, dst, n_pad[kd], ch[kd], zeros_h)
            h[r], stats = _tc_gin_h(feats[kd], agg,
                                    p[r]["W"], _aux_pb(p[r]["b"], p[r]["a"]),
                                    n_real[kd])
            st[r] = _bn_affine(stats, n_real[kd], p[r]["gamma"], p[r]["beta"])
        feats = {
            "a": _tc_affine1(h["aa"], st["aa"]),
            "b": _tc_affine2(h["ab"], st["ab"], h["bb"], st["bb"]),
            "m": _tc_affine2(h["bm"], st["bm"], h["mm"], st["mm"]),
        }

    pad_a = G + 8 + (jnp.arange(nap - na, dtype=jnp.int32) % 16)
    pad_b = G + 8 + (jnp.arange(nbp - nb_, dtype=jnp.int32) % 16)
    pad_m = G + 8 + (jnp.arange(nmp - nm, dtype=jnp.int32) % 16)
    bap = jnp.concatenate([batch_atoms.astype(jnp.int32), pad_a])
    bbp = jnp.concatenate([batch_bonds.astype(jnp.int32), pad_b])
    bmp = jnp.concatenate([batch_monosacchs.astype(jnp.int32), pad_m])

    gsum, gcnt = _sc_pool(feats["a"], bap, feats["b"], bbp, feats["m"], bmp,
                          z128, z16, ones16)

    hd = params["head"]
    w1p = jnp.pad(hd["l1"]["W"], ((0, 0), (0, F - hd["l1"]["W"].shape[1])))
    w2p = jnp.pad(hd["l2"]["W"], ((0, F - hd["l2"]["W"].shape[0]),
                                  (0, F - hd["l2"]["W"].shape[1])))
    auxh = (jnp.zeros((8, F), jnp.float32)
            .at[0, 0:hd["l1"]["b"].shape[0]].set(hd["l1"]["b"])
            .at[1].set(jnp.full((F,), hd["a"], jnp.float32))
            .at[2, 0:hd["l2"]["b"].shape[0]].set(hd["l2"]["b"]))
    pred = _tc_head(gsum, gcnt, w1p, w2p, auxh)
    return pred[:G, 0]


# speedup vs baseline: 1.5629x; 1.3677x over previous
import functools

import jax
import jax.numpy as jnp
from jax import lax
from jax.experimental import pallas as pl
from jax.experimental.pallas import tpu as pltpu
from jax.experimental.pallas import tpu_sc as plsc

F = 128
NCORE = 2
NTILE = 16
WE = 128
WEA = 64
G = 1000
GP = 1024


def _rup(n, m):
    return -(-n // m) * m


def _pad1(x, n, val):
    return jnp.concatenate([x, jnp.full((n - x.shape[0],), val, x.dtype)])


def _dst_cfg(n):
    ch = 12544 if n >= 20000 else 2560
    nch = max(2, _rup(-(-n // ch), 2))
    return nch * ch, ch


def _sc_agg(feats, src, dst, n_dst_p, ch, zeros_h):
    ep = src.shape[0]
    per_tile = ep // NTILE
    nw = per_tile // WEA
    n_chunks = n_dst_p // ch
    cpc = n_chunks // NCORE
    acc_rows = ch + 256
    ptr = acc_rows // NTILE
    zr = zeros_h.shape[0]
    nseg = -(-ptr // zr)
    wpt = ch // NTILE

    mesh = plsc.VectorSubcoreMesh(core_axis_name="c", subcore_axis_name="s")

    @functools.partial(
        pl.kernel, mesh=mesh,
        out_type=jax.ShapeDtypeStruct((n_dst_p, F), jnp.float32),
        scratch_types=[
            pltpu.VMEM_SHARED((acc_rows, F), jnp.float32),
            pltpu.VMEM((4, WEA), jnp.int32),
            pltpu.VMEM((4, WEA), jnp.int32),
            pltpu.VMEM((2, WEA, F), jnp.float32),
            pltpu.VMEM((zeros_h.shape[0], F), jnp.float32),
            [pltpu.SemaphoreType.DMA] * 4,
            [pltpu.SemaphoreType.DMA] * 2,
            [pltpu.SemaphoreType.DMA] * 2,
        ])
    def k(feats_h, src_h, dst_h, z_h, out_h, acc_sh, sidx_v, lidx_v, rows_v,
          zbuf_v, si, sg, ss):
        c = lax.axis_index("c")
        s = lax.axis_index("s")
        lane = lax.iota(jnp.int32, 16)
        pltpu.sync_copy(z_h, zbuf_v)
        e0 = s * per_tile

        def idx_start(kk, bi):
            off = e0 + kk * WEA
            pltpu.async_copy(src_h.at[pl.ds(off, WEA)], sidx_v.at[bi], si[bi])
            pltpu.async_copy(dst_h.at[pl.ds(off, WEA)], lidx_v.at[bi], si[bi])

        def idx_wait(bi):
            pltpu.make_async_copy(src_h.at[pl.ds(0, WEA)], sidx_v.at[bi],
                                  si[bi]).wait()
            pltpu.make_async_copy(dst_h.at[pl.ds(0, WEA)], lidx_v.at[bi],
                                  si[bi]).wait()

        def g_start(bi, rb):
            pltpu.async_copy(feats_h.at[sidx_v.at[bi]], rows_v.at[rb], sg[rb])

        def g_wait(rb):
            pltpu.make_async_copy(feats_h.at[sidx_v.at[0]], rows_v.at[rb],
                                  sg[rb]).wait()

        def s_start(bi, rb):
            pltpu.async_copy(rows_v.at[rb], acc_sh.at[lidx_v.at[bi]], ss[rb],
                             add=True)

        def s_wait(rb):
            pltpu.make_async_copy(rows_v.at[rb], acc_sh.at[lidx_v.at[0]],
                                  ss[rb]).wait()

        for ci in range(cpc):
            chunk = ci * NCORE + c
            base = chunk * ch
            for i in range(nseg):
                st = s * ptr + min(i * zr, ptr - zr)
                pltpu.sync_copy(zbuf_v, acc_sh.at[pl.ds(st, zr)])
            plsc.subcore_barrier()
            dump0 = ch + s * 16

            idx_start(0, 0)
            idx_start(1, 1)

            def _iter(p, _):
                for b in range(4):
                    rb = b % 2
                    kk = 4 * p + b
                    if b < 2:
                        @pl.when(p > 0)
                        def _():
                            s_wait(rb)
                    else:
                        s_wait(rb)

                    @pl.when(kk + 2 < nw)
                    def _():
                        idx_start(kk + 2, (b + 2) % 4)

                    idx_wait(b)
                    for g in range(WEA // 16):
                        dv = lidx_v[b, pl.ds(g * 16, 16)]
                        inm = (dv >= base) & (dv < base + ch)
                        lv = jnp.where(inm, dv - base, dump0 + lane)
                        lidx_v[b, pl.ds(g * 16, 16)] = lv
                    g_start(b, rb)
                    if b == 0:
                        @pl.when(p > 0)
                        def _():
                            g_wait(1)
                            s_start(3, 1)
                    else:
                        g_wait(rb ^ 1)
                        s_start(b - 1, rb ^ 1)
                return 0

            lax.fori_loop(0, nw // 4, _iter, 0)
            g_wait(1)
            s_start(3, 1)
            s_wait(0)
            s_wait(1)
            plsc.subcore_barrier()
            pltpu.sync_copy(acc_sh.at[pl.ds(s * wpt, wpt)],
                            out_h.at[pl.ds(base + s * wpt, wpt)])
            plsc.subcore_barrier()

    return k(feats, src, dst, zeros_h)


def _sc_pool(fa, ba, fb, bb, fm, bm, z128, z16, ones16):
    mesh = plsc.VectorSubcoreMesh(core_axis_name="c", subcore_axis_name="s")
    sizes = (fa.shape[0], fb.shape[0], fm.shape[0])

    @functools.partial(
        pl.kernel, mesh=mesh,
        out_type=(jax.ShapeDtypeStruct((NCORE * GP, F), jnp.float32),
                  jax.ShapeDtypeStruct((NCORE * GP, F), jnp.float32)),
        scratch_types=[
            pltpu.VMEM_SHARED((GP, F), jnp.float32),
            pltpu.VMEM_SHARED((GP, F), jnp.float32),
            pltpu.VMEM((1, WE), jnp.int32),
            pltpu.VMEM((WE, F), jnp.float32),
            pltpu.VMEM((WE, F), jnp.float32),
            pltpu.SemaphoreType.DMA,
        ])
    def k(fa_h, ba_h, fb_h, bb_h, fm_h, bm_h, z128_h, z16_h, ones_h,
          gsum_h, gcnt_h, acc_sh, cacc_sh, idx_v, rows_v, ones_v, sem):
        c = lax.axis_index("c")
        s = lax.axis_index("s")
        wid = s * NCORE + c
        spt = GP // NTILE
        pltpu.sync_copy(z128_h, acc_sh.at[pl.ds(s * spt, spt)])
        pltpu.sync_copy(z128_h, cacc_sh.at[pl.ds(s * spt, spt)])
        pltpu.sync_copy(ones_h, ones_v)
        plsc.subcore_barrier()
        for f_h, b_h, np_ in ((fa_h, ba_h, sizes[0]), (fb_h, bb_h, sizes[1]),
                              (fm_h, bm_h, sizes[2])):
            tot_w = np_ // WE
            nw_max = -(-tot_w // (NCORE * NTILE))

            def _win(j, _, f_h=f_h, b_h=b_h, tot_w=tot_w):
                gw = j * (NCORE * NTILE) + wid

                @pl.when(gw < tot_w)
                def _():
                    off = gw * WE
                    pltpu.sync_copy(b_h.at[pl.ds(off, WE)], idx_v.at[0])
                    pltpu.sync_copy(f_h.at[pl.ds(off, WE)], rows_v)
                    pltpu.sync_copy(rows_v, acc_sh.at[idx_v.at[0]], add=True)
                    pltpu.sync_copy(ones_v, cacc_sh.at[idx_v.at[0]], add=True)

                return 0

            lax.fori_loop(0, nw_max, _win, 0)
        plsc.subcore_barrier()
        pltpu.sync_copy(acc_sh.at[pl.ds(s * spt, spt)],
                        gsum_h.at[pl.ds(c * GP + s * spt, spt)])
        pltpu.sync_copy(cacc_sh.at[pl.ds(s * spt, spt)],
                        gcnt_h.at[pl.ds(c * GP + s * spt, spt)])

    return k(fa, ba, fb, bb, fm, bm, z128, z16, ones16)


def _tc_embed(codes, table):
    n_pad = codes.shape[0]
    br = 1024
    nb = n_pad // br
    cp = table.shape[0]
    c3 = codes.reshape(nb, 1, br)

    def body(c_ref, t_ref, o_ref):
        cod = c_ref[0, 0, :]
        oh = (cod[:, None] == lax.broadcasted_iota(jnp.int32, (br, cp), 1)
              ).astype(jnp.float32)
        o_ref[...] = jnp.dot(oh, t_ref[...], preferred_element_type=jnp.float32)

    return pl.pallas_call(
        body, grid=(nb,),
        in_specs=[pl.BlockSpec((1, 1, br), lambda i: (i, 0, 0)),
                  pl.BlockSpec((cp, F), lambda i: (0, 0))],
        out_specs=pl.BlockSpec((br, F), lambda i: (i, 0)),
        out_shape=jax.ShapeDtypeStruct((n_pad, F), jnp.float32),
    )(c3, table)


def _tc_gin_h(xd, agg, wmat, aux, n_real):
    ndp = xd.shape[0]
    br = 512
    nb = ndp // br

    def body(x_ref, g_ref, w_ref, aux_ref, h_ref, st_ref):
        i = pl.program_id(0)

        @pl.when(i == 0)
        def _():
            st_ref[...] = jnp.zeros_like(st_ref)

        z = jnp.dot(x_ref[...] + g_ref[...], w_ref[...],
                    preferred_element_type=jnp.float32) + aux_ref[0:1, :]
        h = jnp.maximum(z, 0.0) + aux_ref[1:2, :] * jnp.minimum(z, 0.0)
        h_ref[...] = h
        rid = i * br + lax.broadcasted_iota(jnp.int32, (br, 1), 0)
        hm = h * (rid < n_real).astype(jnp.float32)
        s0 = jnp.sum(hm, axis=0, keepdims=True)
        s1 = jnp.sum(hm * hm, axis=0, keepdims=True)
        st_ref[...] += jnp.concatenate(
            [s0, s1, jnp.zeros((6, F), jnp.float32)], axis=0)

    return pl.pallas_call(
        body, grid=(nb,),
        in_specs=[pl.BlockSpec((br, F), lambda i: (i, 0)),
                  pl.BlockSpec((br, F), lambda i: (i, 0)),
                  pl.BlockSpec((F, F), lambda i: (0, 0)),
                  pl.BlockSpec((8, F), lambda i: (0, 0))],
        out_specs=[pl.BlockSpec((br, F), lambda i: (i, 0)),
                   pl.BlockSpec((8, F), lambda i: (0, 0))],
        out_shape=[jax.ShapeDtypeStruct((ndp, F), jnp.float32),
                   jax.ShapeDtypeStruct((8, F), jnp.float32)],
    )(xd, agg, wmat, aux)


def _tc_affine1(h1, st1):
    ndp = h1.shape[0]
    br = 512

    def body(a_ref, sa_ref, o_ref):
        o_ref[...] = a_ref[...] * sa_ref[0:1, :] + sa_ref[1:2, :]

    return pl.pallas_call(
        body, grid=(ndp // br,),
        in_specs=[pl.BlockSpec((br, F), lambda i: (i, 0)),
                  pl.BlockSpec((8, F), lambda i: (0, 0))],
        out_specs=pl.BlockSpec((br, F), lambda i: (i, 0)),
        out_shape=jax.ShapeDtypeStruct((ndp, F), jnp.float32),
    )(h1, st1)


def _tc_affine2(h1, st1, h2, st2):
    ndp = h1.shape[0]
    br = 512

    def body(a_ref, sa_ref, b_ref, sb_ref, o_ref):
        o_ref[...] = (a_ref[...] * sa_ref[0:1, :] + sa_ref[1:2, :]
                      + b_ref[...] * sb_ref[0:1, :] + sb_ref[1:2, :])

    return pl.pallas_call(
        body, grid=(ndp // br,),
        in_specs=[pl.BlockSpec((br, F), lambda i: (i, 0)),
                  pl.BlockSpec((8, F), lambda i: (0, 0)),
                  pl.BlockSpec((br, F), lambda i: (i, 0)),
                  pl.BlockSpec((8, F), lambda i: (0, 0))],
        out_specs=pl.BlockSpec((br, F), lambda i: (i, 0)),
        out_shape=jax.ShapeDtypeStruct((ndp, F), jnp.float32),
    )(h1, st1, h2, st2)


def _tc_head(gsum, gcnt, w1p, w2p, aux):
    def body(gs_ref, gc_ref, w1_ref, w2_ref, aux_ref, o_ref):
        gsum2 = gs_ref[0:GP, :] + gs_ref[GP:2 * GP, :]
        cnt = gc_ref[0:GP, :] + gc_ref[GP:2 * GP, :]
        c1 = jnp.sum(cnt, axis=1, keepdims=True) * (1.0 / F)
        g = gsum2 / jnp.maximum(c1, 1.0)
        z1 = jnp.dot(g, w1_ref[...],
                     preferred_element_type=jnp.float32) + aux_ref[0:1, :]
        h1 = jnp.maximum(z1, 0.0) + aux_ref[1:2, :] * jnp.minimum(z1, 0.0)
        o_ref[...] = jnp.dot(h1, w2_ref[...],
                             preferred_element_type=jnp.float32) + aux_ref[2:3, :]

    return pl.pallas_call(
        body,
        out_shape=jax.ShapeDtypeStruct((GP, F), jnp.float32),
    )(gsum, gcnt, w1p, w2p, aux)


def _bn_affine(st, n, gamma, beta):
    mu = st[0] / n
    var = st[1] / n - mu * mu
    sc = gamma * lax.rsqrt(var + 1e-5)
    tt = beta - mu * sc
    return jnp.zeros((8, F), jnp.float32).at[0].set(sc).at[1].set(tt)


def _aux_pb(b, a):
    return (jnp.zeros((8, F), jnp.float32)
            .at[0].set(b).at[1].set(jnp.full((F,), a, jnp.float32)))


def kernel(x_atoms, x_bonds, x_monosacchs, ei_aa, ei_ab, ei_bb, ei_bm, ei_mm,
           batch_atoms, batch_bonds, batch_monosacchs, params):
    na, nb_, nm = x_atoms.shape[0], x_bonds.shape[0], x_monosacchs.shape[0]
    nap, ch_a = _dst_cfg(na)
    nbp, ch_b = _dst_cfg(nb_)
    nmp, ch_m = _dst_cfg(nm)
    n_real = {"a": na, "b": nb_, "m": nm}
    n_pad = {"a": nap, "b": nbp, "m": nmp}
    ch = {"a": ch_a, "b": ch_b, "m": ch_m}

    zeros_h = jnp.zeros((32, F), jnp.float32)
    z128 = jnp.zeros((GP // NTILE, F), jnp.float32)
    z16 = jnp.zeros((GP // NTILE, 16), jnp.float32)
    ones16 = jnp.ones((WE, F), jnp.float32)

    ta = jnp.pad(params["atom_emb"], ((0, 8), (0, 0)))
    tb = params["bond_emb"]
    tm = params["mono_emb"]
    feats = {
        "a": _tc_embed(_pad1(x_atoms.astype(jnp.int32), nap, 0), ta),
        "b": _tc_embed(_pad1(x_bonds.astype(jnp.int32), nbp, 0), tb),
        "m": _tc_embed(_pad1(x_monosacchs.astype(jnp.int32), nmp, 0), tm),
    }

    def _edges(ei):
        ep = _rup(ei.shape[1], NTILE * WEA * 4)
        return (_pad1(ei[0].astype(jnp.int32), ep, 0),
                _pad1(ei[1].astype(jnp.int32), ep, -1))

    rels = {
        "aa": (_edges(ei_aa), "a", "a"),
        "ab": (_edges(ei_ab), "a", "b"),
        "bb": (_edges(ei_bb), "b", "b"),
        "bm": (_edges(ei_bm), "b", "m"),
        "mm": (_edges(ei_mm), "m", "m"),
    }

    for l in range(3):
        p = params["convs"][l]
        h, st = {}, {}
        for r, ((src, dst), ks, kd) in rels.items():
            agg = _sc_agg(feats[ks], src, dst, n_pad[kd], ch[kd], zeros_h)
            h[r], stats = _tc_gin_h(feats[kd], agg,
                                    p[r]["W"], _aux_pb(p[r]["b"], p[r]["a"]),
                                    n_real[kd])
            st[r] = _bn_affine(stats, n_real[kd], p[r]["gamma"], p[r]["beta"])
        feats = {
            "a": _tc_affine1(h["aa"], st["aa"]),
            "b": _tc_affine2(h["ab"], st["ab"], h["bb"], st["bb"]),
            "m": _tc_affine2(h["bm"], st["bm"], h["mm"], st["mm"]),
        }

    pad_a = G + 8 + (jnp.arange(nap - na, dtype=jnp.int32) % 16)
    pad_b = G + 8 + (jnp.arange(nbp - nb_, dtype=jnp.int32) % 16)
    pad_m = G + 8 + (jnp.arange(nmp - nm, dtype=jnp.int32) % 16)
    bap = jnp.concatenate([batch_atoms.astype(jnp.int32), pad_a])
    bbp = jnp.concatenate([batch_bonds.astype(jnp.int32), pad_b])
    bmp = jnp.concatenate([batch_monosacchs.astype(jnp.int32), pad_m])

    gsum, gcnt = _sc_pool(feats["a"], bap, feats["b"], bbp, feats["m"], bmp,
                          z128, z16, ones16)

    hd = params["head"]
    w1p = jnp.pad(hd["l1"]["W"], ((0, 0), (0, F - hd["l1"]["W"].shape[1])))
    w2p = jnp.pad(hd["l2"]["W"], ((0, F - hd["l2"]["W"].shape[0]),
                                  (0, F - hd["l2"]["W"].shape[1])))
    auxh = (jnp.zeros((8, F), jnp.float32)
            .at[0, 0:hd["l1"]["b"].shape[0]].set(hd["l1"]["b"])
            .at[1].set(jnp.full((F,), hd["a"], jnp.float32))
            .at[2, 0:hd["l2"]["b"].shape[0]].set(hd["l2"]["b"]))
    pred = _tc_head(gsum, gcnt, w1p, w2p, auxh)
    return pred[:G, 0]


# speedup vs baseline: 2.5050x; 1.6028x over previous
import functools

import jax
import jax.numpy as jnp
from jax import lax
from jax.experimental import pallas as pl
from jax.experimental.pallas import tpu as pltpu
from jax.experimental.pallas import tpu_sc as plsc

F = 128
NCORE = 2
NTILE = 16
WE = 128
WEA = 64
G = 1000
GP = 1024


def _rup(n, m):
    return -(-n // m) * m


def _pad1(x, n, val):
    return jnp.concatenate([x, jnp.full((n - x.shape[0],), val, x.dtype)])


def _npad(n):
    return _rup(n, 1024)


def _sc_agg(feats4, src, dst, n_dst_p, zeros_h):
    ep = src.shape[0]
    per_tile = ep // NTILE
    nw = per_tile // WEA
    acc_rows = _rup(n_dst_p + 16, 128)
    ptr = acc_rows // NTILE
    zr = zeros_h.shape[0]
    nseg = -(-ptr // zr)
    wpt = n_dst_p // NTILE

    mesh = plsc.VectorSubcoreMesh(core_axis_name="c", subcore_axis_name="s")

    @functools.partial(
        pl.kernel, mesh=mesh,
        compiler_params=pltpu.CompilerParams(use_tc_tiling_on_sc=False),
        out_type=tuple(jax.ShapeDtypeStruct((n_dst_p, 32), jnp.float32)
                       for _ in range(4)),
        scratch_types=[
            pltpu.VMEM_SHARED((acc_rows, 32), jnp.float32),
            pltpu.VMEM((4, WEA), jnp.int32),
            pltpu.VMEM((4, WEA), jnp.int32),
            pltpu.VMEM((2, WEA, 32), jnp.float32),
            pltpu.VMEM((zeros_h.shape[0], 32), jnp.float32),
            [pltpu.SemaphoreType.DMA] * 4,
            [pltpu.SemaphoreType.DMA] * 2,
            [pltpu.SemaphoreType.DMA] * 2,
        ])
    def k(feats_h, src_h, dst_h, z_h, o0_h, o1_h, o2_h, o3_h, acc_sh, sidx_v,
          lidx_v, rows_v, zbuf_v, si, sg, ss):
        outs = (o0_h, o1_h, o2_h, o3_h)
        c = lax.axis_index("c")
        s = lax.axis_index("s")
        pltpu.sync_copy(z_h, zbuf_v)
        e0 = s * per_tile

        def idx_start(kk, bi):
            off = e0 + kk * WEA
            pltpu.async_copy(src_h.at[pl.ds(off, WEA)], sidx_v.at[bi], si[bi])
            pltpu.async_copy(dst_h.at[pl.ds(off, WEA)], lidx_v.at[bi], si[bi])

        def idx_wait(bi):
            pltpu.make_async_copy(src_h.at[pl.ds(0, WEA)], sidx_v.at[bi],
                                  si[bi]).wait()
            pltpu.make_async_copy(dst_h.at[pl.ds(0, WEA)], lidx_v.at[bi],
                                  si[bi]).wait()

        def g_start(bi, rb):
            pltpu.async_copy(feats_h.at[sidx_v.at[bi]], rows_v.at[rb], sg[rb])

        def g_wait(rb):
            pltpu.make_async_copy(feats_h.at[sidx_v.at[0]], rows_v.at[rb],
                                  sg[rb]).wait()

        def s_start(bi, rb):
            pltpu.async_copy(rows_v.at[rb], acc_sh.at[lidx_v.at[bi]], ss[rb],
                             add=True)

        def s_wait(rb):
            pltpu.make_async_copy(rows_v.at[rb], acc_sh.at[lidx_v.at[0]],
                                  ss[rb]).wait()

        for p4 in range(4):
            @pl.when(c == p4 % 2)
            def _(p4=p4):
                cp = p4
                for i0 in range(0, nseg, 8):
                    for i in range(i0, min(i0 + 8, nseg)):
                        st = s * ptr + min(i * zr, ptr - zr)
                        pltpu.async_copy(zbuf_v, acc_sh.at[pl.ds(st, zr)],
                                         sg[0])
                    for i in range(i0, min(i0 + 8, nseg)):
                        pltpu.make_async_copy(zbuf_v, acc_sh.at[pl.ds(0, zr)],
                                              sg[0]).wait()
                plsc.subcore_barrier()

                idx_start(0, 0)
                idx_start(1, 1)

                def _iter(p, _):
                    for b in range(4):
                        rb = b % 2
                        kk = 4 * p + b
                        if b < 2:
                            @pl.when(p > 0)
                            def _():
                                s_wait(rb)
                        else:
                            s_wait(rb)

                        @pl.when(kk + 2 < nw)
                        def _():
                            idx_start(kk + 2, (b + 2) % 4)

                        idx_wait(b)
                        for g in range(WEA // 16):
                            sv = sidx_v[b, pl.ds(g * 16, 16)]
                            sidx_v[b, pl.ds(g * 16, 16)] = sv * 4 + cp
                        g_start(b, rb)
                        if b == 0:
                            @pl.when(p > 0)
                            def _():
                                g_wait(1)
                                s_start(3, 1)
                        else:
                            g_wait(rb ^ 1)
                            s_start(b - 1, rb ^ 1)
                    return 0

                lax.fori_loop(0, nw // 4, _iter, 0)
                g_wait(1)
                s_start(3, 1)
                s_wait(0)
                s_wait(1)
                plsc.subcore_barrier()
                pltpu.sync_copy(acc_sh.at[pl.ds(s * wpt, wpt)],
                                outs[p4].at[pl.ds(s * wpt, wpt)])
                plsc.subcore_barrier()

    return k(feats4, src, dst, zeros_h)


def _sc_pool(fa, ba, fb, bb, fm, bm, z128, z16, ones16):
    mesh = plsc.VectorSubcoreMesh(core_axis_name="c", subcore_axis_name="s")
    sizes = (fa.shape[0], fb.shape[0], fm.shape[0])

    @functools.partial(
        pl.kernel, mesh=mesh,
        out_type=(jax.ShapeDtypeStruct((NCORE * GP, F), jnp.float32),
                  jax.ShapeDtypeStruct((NCORE * GP, F), jnp.float32)),
        scratch_types=[
            pltpu.VMEM_SHARED((GP, F), jnp.float32),
            pltpu.VMEM_SHARED((GP, F), jnp.float32),
            pltpu.VMEM((1, WE), jnp.int32),
            pltpu.VMEM((WE, F), jnp.float32),
            pltpu.VMEM((WE, F), jnp.float32),
            pltpu.SemaphoreType.DMA,
        ])
    def k(fa_h, ba_h, fb_h, bb_h, fm_h, bm_h, z128_h, z16_h, ones_h,
          gsum_h, gcnt_h, acc_sh, cacc_sh, idx_v, rows_v, ones_v, sem):
        c = lax.axis_index("c")
        s = lax.axis_index("s")
        wid = s * NCORE + c
        spt = GP // NTILE
        pltpu.sync_copy(z128_h, acc_sh.at[pl.ds(s * spt, spt)])
        pltpu.sync_copy(z128_h, cacc_sh.at[pl.ds(s * spt, spt)])
        pltpu.sync_copy(ones_h, ones_v)
        plsc.subcore_barrier()
        for f_h, b_h, np_ in ((fa_h, ba_h, sizes[0]), (fb_h, bb_h, sizes[1]),
                              (fm_h, bm_h, sizes[2])):
            tot_w = np_ // WE
            nw_max = -(-tot_w // (NCORE * NTILE))

            def _win(j, _, f_h=f_h, b_h=b_h, tot_w=tot_w):
                gw = j * (NCORE * NTILE) + wid

                @pl.when(gw < tot_w)
                def _():
                    off = gw * WE
                    pltpu.sync_copy(b_h.at[pl.ds(off, WE)], idx_v.at[0])
                    pltpu.sync_copy(f_h.at[pl.ds(off, WE)], rows_v)
                    pltpu.sync_copy(rows_v, acc_sh.at[idx_v.at[0]], add=True)
                    pltpu.sync_copy(ones_v, cacc_sh.at[idx_v.at[0]], add=True)

                return 0

            lax.fori_loop(0, nw_max, _win, 0)
        plsc.subcore_barrier()
        pltpu.sync_copy(acc_sh.at[pl.ds(s * spt, spt)],
                        gsum_h.at[pl.ds(c * GP + s * spt, spt)])
        pltpu.sync_copy(cacc_sh.at[pl.ds(s * spt, spt)],
                        gcnt_h.at[pl.ds(c * GP + s * spt, spt)])

    return k(fa, ba, fb, bb, fm, bm, z128, z16, ones16)


def _tc_embed(codes, table):
    n_pad = codes.shape[0]
    br = 1024
    nb = n_pad // br
    cp = table.shape[0]
    c3 = codes.reshape(nb, 1, br)

    def body(c_ref, t_ref, o_ref):
        cod = c_ref[0, 0, :]
        oh = (cod[:, None] == lax.broadcasted_iota(jnp.int32, (br, cp), 1)
              ).astype(jnp.float32)
        o_ref[...] = jnp.dot(oh, t_ref[...], preferred_element_type=jnp.float32)

    return pl.pallas_call(
        body, grid=(nb,),
        in_specs=[pl.BlockSpec((1, 1, br), lambda i: (i, 0, 0)),
                  pl.BlockSpec((cp, F), lambda i: (0, 0))],
        out_specs=pl.BlockSpec((br, F), lambda i: (i, 0)),
        out_shape=jax.ShapeDtypeStruct((n_pad, F), jnp.float32),
    )(c3, table)


def _tc_gin_h(xd, agg4, wmat, aux, n_real):
    ndp = xd.shape[0]
    br = 512
    nb = ndp // br

    def body(x_ref, g0_ref, g1_ref, g2_ref, g3_ref, w_ref, aux_ref, h_ref,
             st_ref):
        i = pl.program_id(0)

        @pl.when(i == 0)
        def _():
            st_ref[...] = jnp.zeros_like(st_ref)

        gg = jnp.concatenate([g0_ref[...], g1_ref[...], g2_ref[...],
                              g3_ref[...]], axis=1)
        z = jnp.dot(x_ref[...] + gg, w_ref[...],
                    preferred_element_type=jnp.float32) + aux_ref[0:1, :]
        h = jnp.maximum(z, 0.0) + aux_ref[1:2, :] * jnp.minimum(z, 0.0)
        h_ref[...] = h
        rid = i * br + lax.broadcasted_iota(jnp.int32, (br, 1), 0)
        hm = h * (rid < n_real).astype(jnp.float32)
        s0 = jnp.sum(hm, axis=0, keepdims=True)
        s1 = jnp.sum(hm * hm, axis=0, keepdims=True)
        st_ref[...] += jnp.concatenate(
            [s0, s1, jnp.zeros((6, F), jnp.float32)], axis=0)

    return pl.pallas_call(
        body, grid=(nb,),
        in_specs=[pl.BlockSpec((br, F), lambda i: (i, 0)),
                  pl.BlockSpec((br, 32), lambda i: (i, 0)),
                  pl.BlockSpec((br, 32), lambda i: (i, 0)),
                  pl.BlockSpec((br, 32), lambda i: (i, 0)),
                  pl.BlockSpec((br, 32), lambda i: (i, 0)),
                  pl.BlockSpec((F, F), lambda i: (0, 0)),
                  pl.BlockSpec((8, F), lambda i: (0, 0))],
        out_specs=[pl.BlockSpec((br, F), lambda i: (i, 0)),
                   pl.BlockSpec((8, F), lambda i: (0, 0))],
        out_shape=[jax.ShapeDtypeStruct((ndp, F), jnp.float32),
                   jax.ShapeDtypeStruct((8, F), jnp.float32)],
    )(xd, agg4[0], agg4[1], agg4[2], agg4[3], wmat, aux)


def _tc_affine1(h1, st1):
    ndp = h1.shape[0]
    br = 512

    def body(a_ref, sa_ref, o_ref):
        o_ref[...] = a_ref[...] * sa_ref[0:1, :] + sa_ref[1:2, :]

    return pl.pallas_call(
        body, grid=(ndp // br,),
        in_specs=[pl.BlockSpec((br, F), lambda i: (i, 0)),
                  pl.BlockSpec((8, F), lambda i: (0, 0))],
        out_specs=pl.BlockSpec((br, F), lambda i: (i, 0)),
        out_shape=jax.ShapeDtypeStruct((ndp, F), jnp.float32),
    )(h1, st1)


def _tc_affine2(h1, st1, h2, st2):
    ndp = h1.shape[0]
    br = 512

    def body(a_ref, sa_ref, b_ref, sb_ref, o_ref):
        o_ref[...] = (a_ref[...] * sa_ref[0:1, :] + sa_ref[1:2, :]
                      + b_ref[...] * sb_ref[0:1, :] + sb_ref[1:2, :])

    return pl.pallas_call(
        body, grid=(ndp // br,),
        in_specs=[pl.BlockSpec((br, F), lambda i: (i, 0)),
                  pl.BlockSpec((8, F), lambda i: (0, 0)),
                  pl.BlockSpec((br, F), lambda i: (i, 0)),
                  pl.BlockSpec((8, F), lambda i: (0, 0))],
        out_specs=pl.BlockSpec((br, F), lambda i: (i, 0)),
        out_shape=jax.ShapeDtypeStruct((ndp, F), jnp.float32),
    )(h1, st1, h2, st2)


def _tc_head(gsum, gcnt, w1p, w2p, aux):
    def body(gs_ref, gc_ref, w1_ref, w2_ref, aux_ref, o_ref):
        gsum2 = gs_ref[0:GP, :] + gs_ref[GP:2 * GP, :]
        cnt = gc_ref[0:GP, :] + gc_ref[GP:2 * GP, :]
        c1 = jnp.sum(cnt, axis=1, keepdims=True) * (1.0 / F)
        g = gsum2 / jnp.maximum(c1, 1.0)
        z1 = jnp.dot(g, w1_ref[...],
                     preferred_element_type=jnp.float32) + aux_ref[0:1, :]
        h1 = jnp.maximum(z1, 0.0) + aux_ref[1:2, :] * jnp.minimum(z1, 0.0)
        o_ref[...] = jnp.dot(h1, w2_ref[...],
                             preferred_element_type=jnp.float32) + aux_ref[2:3, :]

    return pl.pallas_call(
        body,
        out_shape=jax.ShapeDtypeStruct((GP, F), jnp.float32),
    )(gsum, gcnt, w1p, w2p, aux)


def _bn_affine(st, n, gamma, beta):
    mu = st[0] / n
    var = st[1] / n - mu * mu
    sc = gamma * lax.rsqrt(var + 1e-5)
    tt = beta - mu * sc
    return jnp.zeros((8, F), jnp.float32).at[0].set(sc).at[1].set(tt)


def _aux_pb(b, a):
    return (jnp.zeros((8, F), jnp.float32)
            .at[0].set(b).at[1].set(jnp.full((F,), a, jnp.float32)))


def kernel(x_atoms, x_bonds, x_monosacchs, ei_aa, ei_ab, ei_bb, ei_bm, ei_mm,
           batch_atoms, batch_bonds, batch_monosacchs, params):
    na, nb_, nm = x_atoms.shape[0], x_bonds.shape[0], x_monosacchs.shape[0]
    nap, nbp, nmp = _npad(na), _npad(nb_), _npad(nm)
    n_real = {"a": na, "b": nb_, "m": nm}
    n_pad = {"a": nap, "b": nbp, "m": nmp}

    zeros_h = jnp.zeros((64, 32), jnp.float32)
    z128 = jnp.zeros((GP // NTILE, F), jnp.float32)
    z16 = jnp.zeros((GP // NTILE, 16), jnp.float32)
    ones16 = jnp.ones((WE, F), jnp.float32)

    ta = jnp.pad(params["atom_emb"], ((0, 8), (0, 0)))
    tb = params["bond_emb"]
    tm = params["mono_emb"]
    feats = {
        "a": _tc_embed(_pad1(x_atoms.astype(jnp.int32), nap, 0), ta),
        "b": _tc_embed(_pad1(x_bonds.astype(jnp.int32), nbp, 0), tb),
        "m": _tc_embed(_pad1(x_monosacchs.astype(jnp.int32), nmp, 0), tm),
    }

    def _edges(ei, ndp):
        ep = _rup(ei.shape[1], NTILE * WEA * 4)
        npd = ep - ei.shape[1]
        dpad = ndp + (jnp.arange(npd, dtype=jnp.int32) % 16)
        return (_pad1(ei[0].astype(jnp.int32), ep, 0),
                jnp.concatenate([ei[1].astype(jnp.int32), dpad]))

    rels = {
        "aa": (_edges(ei_aa, nap), "a", "a"),
        "ab": (_edges(ei_ab, nbp), "a", "b"),
        "bb": (_edges(ei_bb, nbp), "b", "b"),
        "bm": (_edges(ei_bm, nmp), "b", "m"),
        "mm": (_edges(ei_mm, nmp), "m", "m"),
    }

    for l in range(3):
        p = params["convs"][l]
        h, st = {}, {}
        for r, ((src, dst), ks, kd) in rels.items():
            agg4 = _sc_agg(feats[ks].reshape(-1, 32), src, dst, n_pad[kd],
                           zeros_h)
            h[r], stats = _tc_gin_h(feats[kd], agg4,
                                    p[r]["W"], _aux_pb(p[r]["b"], p[r]["a"]),
                                    n_real[kd])
            st[r] = _bn_affine(stats, n_real[kd], p[r]["gamma"], p[r]["beta"])
        feats = {
            "a": _tc_affine1(h["aa"], st["aa"]),
            "b": _tc_affine2(h["ab"], st["ab"], h["bb"], st["bb"]),
            "m": _tc_affine2(h["bm"], st["bm"], h["mm"], st["mm"]),
        }

    pad_a = G + 8 + (jnp.arange(nap - na, dtype=jnp.int32) % 16)
    pad_b = G + 8 + (jnp.arange(nbp - nb_, dtype=jnp.int32) % 16)
    pad_m = G + 8 + (jnp.arange(nmp - nm, dtype=jnp.int32) % 16)
    bap = jnp.concatenate([batch_atoms.astype(jnp.int32), pad_a])
    bbp = jnp.concatenate([batch_bonds.astype(jnp.int32), pad_b])
    bmp = jnp.concatenate([batch_monosacchs.astype(jnp.int32), pad_m])

    gsum, gcnt = _sc_pool(feats["a"], bap, feats["b"], bbp, feats["m"], bmp,
                          z128, z16, ones16)

    hd = params["head"]
    w1p = jnp.pad(hd["l1"]["W"], ((0, 0), (0, F - hd["l1"]["W"].shape[1])))
    w2p = jnp.pad(hd["l2"]["W"], ((0, F - hd["l2"]["W"].shape[0]),
                                  (0, F - hd["l2"]["W"].shape[1])))
    auxh = (jnp.zeros((8, F), jnp.float32)
            .at[0, 0:hd["l1"]["b"].shape[0]].set(hd["l1"]["b"])
            .at[1].set(jnp.full((F,), hd["a"], jnp.float32))
            .at[2, 0:hd["l2"]["b"].shape[0]].set(hd["l2"]["b"]))
    pred = _tc_head(gsum, gcnt, w1p, w2p, auxh)
    return pred[:G, 0]
